# baseline (speedup 1.0000x reference)
import jax
import jax.numpy as jnp
from jax import lax
from jax.experimental import pallas as pl
from jax.experimental.pallas import tpu as pltpu
from jax.experimental.pallas import tpu_sc as plsc

N = 10000
E = 320000
D_IN = 128
D_HID = 64
D_OUT = 4

NC = 2
NS = 16
NW = NC * NS
EPAD = 327680
EPW = EPAD // NW
CHUNK = 128
NCHUNK = EPW // CHUNK
NPAD = 10240
SEG = NPAD // NS
ROWS_PT = NPAD // NS
OUTP = 128


def _mesh():
    return plsc.VectorSubcoreMesh(core_axis_name="c", subcore_axis_name="s")


def _deg_body(dst_hbm, zeros_hbm, deg_out, hist_v, idx_v, seg_v, out_v,
              shared_sm):
    c = lax.axis_index("c")
    s = lax.axis_index("s")
    wid = c * NS + s
    pltpu.sync_copy(zeros_hbm, hist_v)
    pltpu.sync_copy(dst_hbm.at[wid], idx_v)
    ones = jnp.full((16,), 1.0, jnp.float32)

    def body(i, carry):
        idx = idx_v[pl.ds(i * 16, 16)]
        plsc.addupdate_scatter(hist_v, [idx], ones)
        return carry

    lax.fori_loop(0, EPW // 16, body, 0)

    for t in range(NS):
        pltpu.sync_copy(hist_v.at[pl.ds(t * SEG, SEG)], shared_sm.at[s, t])
    plsc.subcore_barrier()
    for t in range(NS):
        pltpu.sync_copy(shared_sm.at[t, s], seg_v.at[t])

    def red(k, carry):
        v = jnp.zeros((16,), jnp.float32)
        for t in range(NS):
            v = v + seg_v[t, pl.ds(k * 16, 16)]
        out_v[pl.ds(k * 16, 16)] = v
        return carry

    lax.fori_loop(0, SEG // 16, red, 0)
    pltpu.sync_copy(out_v, deg_out.at[c, pl.ds(s * SEG, SEG)])


_deg = pl.kernel(
    _deg_body,
    out_type=jax.ShapeDtypeStruct((NC, NPAD), jnp.float32),
    mesh=_mesh(),
    scratch_types=[
        pltpu.VMEM((NPAD,), jnp.float32),
        pltpu.VMEM((EPW,), jnp.int32),
        pltpu.VMEM((NS, SEG), jnp.float32),
        pltpu.VMEM((SEG,), jnp.float32),
        pltpu.MemorySpace.VMEM_SHARED((NS, NS, SEG), jnp.float32),
    ],
    compiler_params=pltpu.CompilerParams(needs_layout_passes=False),
)


NB = 5
NGROUP = NCHUNK // NB


def _mp_body(hs_hbm, src_hbm, dst_hbm, zeros_hbm, out_hbm, isrc_v, idst_v,
             rows_v, acc_sm, gsem, ssem):
    c = lax.axis_index("c")
    s = lax.axis_index("s")
    wid = c * NS + s
    r0 = s * ROWS_PT
    pltpu.sync_copy(zeros_hbm.at[pl.ds(r0, ROWS_PT)],
                    acc_sm.at[pl.ds(r0, ROWS_PT)])
    pltpu.sync_copy(src_hbm.at[wid], isrc_v)
    pltpu.sync_copy(dst_hbm.at[wid], idst_v)
    plsc.subcore_barrier()

    for b in range(NB):
        pltpu.async_copy(hs_hbm.at[isrc_v.at[b]], rows_v.at[b], gsem.at[b])

    def body(g, carry):
        for b in range(NB):
            j = g * NB + b
            pltpu.make_async_copy(hs_hbm.at[isrc_v.at[j]], rows_v.at[b],
                                  gsem.at[b]).wait()
            pltpu.async_copy(rows_v.at[b], acc_sm.at[idst_v.at[j]],
                             ssem.at[b], add=True)

        @pl.when(g < NGROUP - 1)
        def _():
            for b in range(NB):
                j = g * NB + b
                jn = j + NB
                pltpu.make_async_copy(rows_v.at[b],
                                      acc_sm.at[idst_v.at[j]],
                                      ssem.at[b]).wait()
                pltpu.async_copy(hs_hbm.at[isrc_v.at[jn]], rows_v.at[b],
                                 gsem.at[b])
        return carry

    lax.fori_loop(0, NGROUP, body, 0)
    for b in range(NB):
        j = (NGROUP - 1) * NB + b
        pltpu.make_async_copy(rows_v.at[b], acc_sm.at[idst_v.at[j]],
                              ssem.at[b]).wait()
    plsc.subcore_barrier()
    pltpu.sync_copy(acc_sm.at[pl.ds(r0, ROWS_PT)],
                    out_hbm.at[c, pl.ds(r0, ROWS_PT)])


_mp = pl.kernel(
    _mp_body,
    out_type=jax.ShapeDtypeStruct((NC, NPAD, D_HID), jnp.float32),
    mesh=_mesh(),
    scratch_types=[
        pltpu.VMEM((NCHUNK, CHUNK), jnp.int32),
        pltpu.VMEM((NCHUNK, CHUNK), jnp.int32),
        pltpu.VMEM((NB, CHUNK, D_HID), jnp.float32),
        pltpu.MemorySpace.VMEM_SHARED((NPAD, D_HID), jnp.float32),
        pltpu.SemaphoreType.DMA((NB,)),
        pltpu.SemaphoreType.DMA((NB,)),
    ],
    compiler_params=pltpu.CompilerParams(needs_layout_passes=False,
                                         use_tc_tiling_on_sc=False),
)


def _tc1_body(x_ref, w_ref, dp_ref, hs_ref, dinv_ref):
    deg = dp_ref[0] + dp_ref[1] + 1.0
    dinv = lax.rsqrt(deg)
    h = jnp.dot(x_ref[...], w_ref[...], preferred_element_type=jnp.float32)
    hs_ref[pl.ds(0, N), :] = h * lax.slice(dinv, (0, 0), (N, 1))
    dinv_ref[...] = dinv


_tc1 = pl.pallas_call(
    _tc1_body,
    in_specs=[
        pl.BlockSpec((N, D_IN), lambda: (0, 0)),
        pl.BlockSpec((D_IN, D_HID), lambda: (0, 0)),
        pl.BlockSpec((NC, NPAD, 1), lambda: (0, 0, 0)),
    ],
    out_specs=[
        pl.BlockSpec((NPAD, D_HID), lambda: (0, 0)),
        pl.BlockSpec((NPAD, 1), lambda: (0, 0)),
    ],
    out_shape=[
        jax.ShapeDtypeStruct((NPAD, D_HID), jnp.float32),
        jax.ShapeDtypeStruct((NPAD, 1), jnp.float32),
    ],
)


def _tc2_body(acc_ref, hs_ref, dinv_ref, b_ref, w_ref, out_ref):
    dinv = dinv_ref[...]
    pre = (acc_ref[0] + acc_ref[1] + hs_ref[...]) * dinv + b_ref[...]
    t = jnp.maximum(pre, 0.0)
    out_ref[...] = jnp.dot(t, w_ref[...],
                           preferred_element_type=jnp.float32) * dinv


_tc2 = pl.pallas_call(
    _tc2_body,
    in_specs=[
        pl.BlockSpec((NC, NPAD, D_HID), lambda: (0, 0, 0)),
        pl.BlockSpec((NPAD, D_HID), lambda: (0, 0)),
        pl.BlockSpec((NPAD, 1), lambda: (0, 0)),
        pl.BlockSpec((1, D_HID), lambda: (0, 0)),
        pl.BlockSpec((D_HID, D_HID), lambda: (0, 0)),
    ],
    out_specs=pl.BlockSpec((NPAD, D_HID), lambda: (0, 0)),
    out_shape=jax.ShapeDtypeStruct((NPAD, D_HID), jnp.float32),
)


def _tc3_body(acc_ref, hs_ref, dinv_ref, b_ref, w_ref, bf_ref, out_ref):
    dinv = dinv_ref[...]
    pre = (acc_ref[0] + acc_ref[1] + hs_ref[...]) * dinv + b_ref[...]
    t = jnp.maximum(pre, 0.0)
    tn = lax.slice(t, (0, 0), (N, D_HID))
    out_ref[...] = jnp.dot(tn, w_ref[...],
                           preferred_element_type=jnp.float32) + bf_ref[...]


_tc3 = pl.pallas_call(
    _tc3_body,
    in_specs=[
        pl.BlockSpec((NC, NPAD, D_HID), lambda: (0, 0, 0)),
        pl.BlockSpec((NPAD, D_HID), lambda: (0, 0)),
        pl.BlockSpec((NPAD, 1), lambda: (0, 0)),
        pl.BlockSpec((1, D_HID), lambda: (0, 0)),
        pl.BlockSpec((D_HID, D_OUT), lambda: (0, 0)),
        pl.BlockSpec((1, D_OUT), lambda: (0, 0)),
    ],
    out_specs=pl.BlockSpec((N, D_OUT), lambda: (0, 0)),
    out_shape=jax.ShapeDtypeStruct((N, D_OUT), jnp.float32),
)


@jax.jit
def kernel(x, edge_index, W1, b1, W2, b2, Wf, bf):
    npad1 = jnp.full((EPAD - E,), NPAD - 1, jnp.int32)
    src = jnp.concatenate([edge_index[0].astype(jnp.int32), npad1])
    dst = jnp.concatenate([edge_index[1].astype(jnp.int32), npad1])
    src = src.reshape(NW, NCHUNK, CHUNK)
    dst = dst.reshape(NW, NCHUNK, CHUNK)
    dst_flat = dst.reshape(NW, EPW)
    zpad = jnp.zeros((NPAD,), jnp.float32)
    znd = jnp.zeros((NPAD, D_HID), jnp.float32)

    deg_parts = _deg(dst_flat, zpad)

    hs1, dinv = _tc1(x, W1, deg_parts.reshape(NC, NPAD, 1))
    acc1 = _mp(hs1, src, dst, znd)
    hs2 = _tc2(acc1, hs1, dinv, b1.reshape(1, D_HID), W2)
    acc2 = _mp(hs2, src, dst, znd)
    return _tc3(acc2, hs2, dinv, b2.reshape(1, D_HID), Wf,
                bf.reshape(1, D_OUT))


# speedup vs baseline: 2.9863x; 2.9863x over previous
import jax
import jax.numpy as jnp
from jax import lax
from jax.experimental import pallas as pl
from jax.experimental.pallas import tpu as pltpu
from jax.experimental.pallas import tpu_sc as plsc

N = 10000
E = 320000
D_IN = 128
D_HID = 64
D_OUT = 4

NC = 2
NS = 16
NW = NC * NS
EPW = E // NW
CHUNK = 80
NCHUNK = EPW // CHUNK
NPAD = 10240
SEG = NPAD // NS
ROWS_PT = NPAD // NS
OUTP = 128


def _mesh():
    return plsc.VectorSubcoreMesh(core_axis_name="c", subcore_axis_name="s")


def _deg_body(dst_hbm, zeros_hbm, deg_out, hist_v, idx_v, seg_v, out_v,
              shared_sm):
    c = lax.axis_index("c")
    s = lax.axis_index("s")
    wid = c * NS + s
    pltpu.sync_copy(zeros_hbm, hist_v)
    pltpu.sync_copy(dst_hbm.at[wid], idx_v)
    ones = jnp.full((16,), 1.0, jnp.float32)

    def body(i, carry):
        idx = idx_v[pl.ds(i * 16, 16)]
        plsc.addupdate_scatter(hist_v, [idx], ones)
        return carry

    lax.fori_loop(0, EPW // 16, body, 0)

    for t in range(NS):
        pltpu.sync_copy(hist_v.at[pl.ds(t * SEG, SEG)], shared_sm.at[s, t])
    plsc.subcore_barrier()
    for t in range(NS):
        pltpu.sync_copy(shared_sm.at[t, s], seg_v.at[t])

    def red(k, carry):
        v = jnp.zeros((16,), jnp.float32)
        for t in range(NS):
            v = v + seg_v[t, pl.ds(k * 16, 16)]
        out_v[pl.ds(k * 16, 16)] = v
        return carry

    lax.fori_loop(0, SEG // 16, red, 0)
    pltpu.sync_copy(out_v, deg_out.at[c, pl.ds(s * SEG, SEG)])


_deg = pl.kernel(
    _deg_body,
    out_type=jax.ShapeDtypeStruct((NC, NPAD), jnp.float32),
    mesh=_mesh(),
    scratch_types=[
        pltpu.VMEM((NPAD,), jnp.float32),
        pltpu.VMEM((EPW,), jnp.int32),
        pltpu.VMEM((NS, SEG), jnp.float32),
        pltpu.VMEM((SEG,), jnp.float32),
        pltpu.MemorySpace.VMEM_SHARED((NS, NS, SEG), jnp.float32),
    ],
    compiler_params=pltpu.CompilerParams(needs_layout_passes=False),
)


NB = 5
NGROUP = NCHUNK // NB


def _mp_body(hs_hbm, src_hbm, dst_hbm, zeros_hbm, out_hbm, isrc_v, idst_v,
             rows_v, acc_sm, gsem, ssem):
    c = lax.axis_index("c")
    s = lax.axis_index("s")
    wid = c * NS + s
    r0 = s * ROWS_PT
    pltpu.sync_copy(zeros_hbm.at[pl.ds(r0, ROWS_PT)],
                    acc_sm.at[pl.ds(r0, ROWS_PT)])
    pltpu.sync_copy(src_hbm.at[wid], isrc_v)
    pltpu.sync_copy(dst_hbm.at[wid], idst_v)
    plsc.subcore_barrier()

    for b in range(NB):
        pltpu.async_copy(hs_hbm.at[isrc_v.at[b]], rows_v.at[b], gsem.at[b])

    def body(g, carry):
        for b in range(NB):
            j = g * NB + b
            pltpu.make_async_copy(hs_hbm.at[isrc_v.at[j]], rows_v.at[b],
                                  gsem.at[b]).wait()
            pltpu.async_copy(rows_v.at[b], acc_sm.at[idst_v.at[j]],
                             ssem.at[b], add=True)

        @pl.when(g < NGROUP - 1)
        def _():
            for b in range(NB):
                j = g * NB + b
                jn = j + NB
                pltpu.make_async_copy(rows_v.at[b],
                                      acc_sm.at[idst_v.at[j]],
                                      ssem.at[b]).wait()
                pltpu.async_copy(hs_hbm.at[isrc_v.at[jn]], rows_v.at[b],
                                 gsem.at[b])
        return carry

    lax.fori_loop(0, NGROUP, body, 0)
    for b in range(NB):
        j = (NGROUP - 1) * NB + b
        pltpu.make_async_copy(rows_v.at[b], acc_sm.at[idst_v.at[j]],
                              ssem.at[b]).wait()
    plsc.subcore_barrier()
    pltpu.sync_copy(acc_sm.at[pl.ds(r0, ROWS_PT)],
                    out_hbm.at[c, pl.ds(r0, ROWS_PT)])


_mp = pl.kernel(
    _mp_body,
    out_type=jax.ShapeDtypeStruct((NC, NPAD, D_HID), jnp.float32),
    mesh=_mesh(),
    scratch_types=[
        pltpu.VMEM((NCHUNK, CHUNK), jnp.int32),
        pltpu.VMEM((NCHUNK, CHUNK), jnp.int32),
        pltpu.VMEM((NB, CHUNK, D_HID), jnp.float32),
        pltpu.MemorySpace.VMEM_SHARED((NPAD, D_HID), jnp.float32),
        pltpu.SemaphoreType.DMA((NB,)),
        pltpu.SemaphoreType.DMA((NB,)),
    ],
    compiler_params=pltpu.CompilerParams(needs_layout_passes=False,
                                         use_tc_tiling_on_sc=False),
)


def _tc1_body(x_ref, w_ref, dp_ref, hs_ref, dinv_ref):
    deg = dp_ref[0] + dp_ref[1] + 1.0
    dinv = lax.rsqrt(deg)
    h = jnp.dot(x_ref[...], w_ref[...], preferred_element_type=jnp.float32)
    hs_ref[pl.ds(0, N), :] = h * lax.slice(dinv, (0, 0), (N, 1))
    dinv_ref[...] = dinv


_tc1 = pl.pallas_call(
    _tc1_body,
    in_specs=[
        pl.BlockSpec((N, D_IN), lambda: (0, 0)),
        pl.BlockSpec((D_IN, D_HID), lambda: (0, 0)),
        pl.BlockSpec((NC, NPAD, 1), lambda: (0, 0, 0)),
    ],
    out_specs=[
        pl.BlockSpec((NPAD, D_HID), lambda: (0, 0)),
        pl.BlockSpec((NPAD, 1), lambda: (0, 0)),
    ],
    out_shape=[
        jax.ShapeDtypeStruct((NPAD, D_HID), jnp.float32),
        jax.ShapeDtypeStruct((NPAD, 1), jnp.float32),
    ],
)


def _tc2_body(acc_ref, hs_ref, dinv_ref, b_ref, w_ref, out_ref):
    dinv = dinv_ref[...]
    pre = (acc_ref[0] + acc_ref[1] + hs_ref[...]) * dinv + b_ref[...]
    t = jnp.maximum(pre, 0.0)
    out_ref[...] = jnp.dot(t, w_ref[...],
                           preferred_element_type=jnp.float32) * dinv


_tc2 = pl.pallas_call(
    _tc2_body,
    in_specs=[
        pl.BlockSpec((NC, NPAD, D_HID), lambda: (0, 0, 0)),
        pl.BlockSpec((NPAD, D_HID), lambda: (0, 0)),
        pl.BlockSpec((NPAD, 1), lambda: (0, 0)),
        pl.BlockSpec((1, D_HID), lambda: (0, 0)),
        pl.BlockSpec((D_HID, D_HID), lambda: (0, 0)),
    ],
    out_specs=pl.BlockSpec((NPAD, D_HID), lambda: (0, 0)),
    out_shape=jax.ShapeDtypeStruct((NPAD, D_HID), jnp.float32),
)


def _tc3_body(acc_ref, hs_ref, dinv_ref, b_ref, w_ref, bf_ref, out_ref):
    dinv = dinv_ref[...]
    pre = (acc_ref[0] + acc_ref[1] + hs_ref[...]) * dinv + b_ref[...]
    t = jnp.maximum(pre, 0.0)
    tn = lax.slice(t, (0, 0), (N, D_HID))
    out_ref[...] = jnp.dot(tn, w_ref[...],
                           preferred_element_type=jnp.float32) + bf_ref[...]


_tc3 = pl.pallas_call(
    _tc3_body,
    in_specs=[
        pl.BlockSpec((NC, NPAD, D_HID), lambda: (0, 0, 0)),
        pl.BlockSpec((NPAD, D_HID), lambda: (0, 0)),
        pl.BlockSpec((NPAD, 1), lambda: (0, 0)),
        pl.BlockSpec((1, D_HID), lambda: (0, 0)),
        pl.BlockSpec((D_HID, D_OUT), lambda: (0, 0)),
        pl.BlockSpec((1, D_OUT), lambda: (0, 0)),
    ],
    out_specs=pl.BlockSpec((N, D_OUT), lambda: (0, 0)),
    out_shape=jax.ShapeDtypeStruct((N, D_OUT), jnp.float32),
)


@jax.jit
def kernel(x, edge_index, W1, b1, W2, b2, Wf, bf):
    src = edge_index[0].astype(jnp.int32).reshape(NW, NCHUNK, CHUNK)
    dst = edge_index[1].astype(jnp.int32).reshape(NW, NCHUNK, CHUNK)
    dst_flat = dst.reshape(NW, EPW)
    zpad = jnp.zeros((NPAD,), jnp.float32)
    znd = jnp.zeros((NPAD, D_HID), jnp.float32)

    deg_parts = _deg(dst_flat, zpad)

    hs1, dinv = _tc1(x, W1, deg_parts.reshape(NC, NPAD, 1))
    acc1 = _mp(hs1, src, dst, znd)
    hs2 = _tc2(acc1, hs1, dinv, b1.reshape(1, D_HID), W2)
    acc2 = _mp(hs2, src, dst, znd)
    return _tc3(acc2, hs2, dinv, b2.reshape(1, D_HID), Wf,
                bf.reshape(1, D_OUT))


# speedup vs baseline: 3.7251x; 1.2474x over previous
import jax
import jax.numpy as jnp
from jax import lax
from jax.experimental import pallas as pl
from jax.experimental.pallas import tpu as pltpu
from jax.experimental.pallas import tpu_sc as plsc

N = 10000
E = 320000
D_IN = 128
D_HID = 64
D_OUT = 4

NC = 2
NS = 16
NW = NC * NS
EPW = E // NW
CHUNK = 80
NCHUNK = EPW // CHUNK
NPAD = 10240
SEG = NPAD // NS
ROWS_PT = NPAD // NS
NPAIR = NPAD // 2
NPAIR_N = N // 2


def _mesh():
    return plsc.VectorSubcoreMesh(core_axis_name="c", subcore_axis_name="s")


def _deg_body(ei_hbm, zeros_hbm, deg_out, hist_v, idx_v, seg_v, out_v,
              shared_sm):
    c = lax.axis_index("c")
    s = lax.axis_index("s")
    wid = c * NS + s
    pltpu.sync_copy(zeros_hbm, hist_v)
    pltpu.sync_copy(ei_hbm.at[1, wid], idx_v)
    ones = jnp.full((16,), 1.0, jnp.float32)

    def body(i, carry):
        for k in range(5):
            idx = idx_v[pl.ds(i * 80 + k * 16, 16)]
            plsc.addupdate_scatter(hist_v, [idx], ones)
        return carry

    lax.fori_loop(0, EPW // 80, body, 0)

    for t in range(NS):
        pltpu.sync_copy(hist_v.at[pl.ds(t * SEG, SEG)], shared_sm.at[s, t])
    plsc.subcore_barrier()
    for t in range(NS):
        pltpu.sync_copy(shared_sm.at[t, s], seg_v.at[t])

    def red(k, carry):
        v = jnp.zeros((16,), jnp.float32)
        for t in range(NS):
            v = v + seg_v[t, pl.ds(k * 16, 16)]
        out_v[pl.ds(k * 16, 16)] = v
        return carry

    lax.fori_loop(0, SEG // 16, red, 0)
    pltpu.sync_copy(out_v, deg_out.at[c, pl.ds(s * SEG, SEG)])


_deg = pl.kernel(
    _deg_body,
    out_type=jax.ShapeDtypeStruct((NC, NPAD), jnp.float32),
    mesh=_mesh(),
    scratch_types=[
        pltpu.VMEM((NPAD,), jnp.float32),
        pltpu.VMEM((EPW,), jnp.int32),
        pltpu.VMEM((NS, SEG), jnp.float32),
        pltpu.VMEM((SEG,), jnp.float32),
        pltpu.MemorySpace.VMEM_SHARED((NS, NS, SEG), jnp.float32),
    ],
    compiler_params=pltpu.CompilerParams(needs_layout_passes=False,
                                         use_tc_tiling_on_sc=False),
)


NB = 5
NGROUP = NCHUNK // NB


def _mp_body(hs_hbm, ei_hbm, out_hbm, isrc_v, idst_v, rows_v, acc_sm,
             gsem, ssem):
    c = lax.axis_index("c")
    s = lax.axis_index("s")
    wid = c * NS + s
    r0 = s * ROWS_PT

    @pl.when(c == 0)
    def _():
        pltpu.sync_copy(hs_hbm.at[pl.ds(r0, ROWS_PT)],
                        acc_sm.at[pl.ds(r0, ROWS_PT)])

    @pl.when(c == 1)
    def _():
        z16 = jnp.zeros((16,), jnp.float32)

        def zero(i, carry):
            for k in range(D_HID // 16):
                rows_v[0, i, pl.ds(k * 16, 16)] = z16
            return carry
        lax.fori_loop(0, CHUNK, zero, 0)
        for q in range(ROWS_PT // CHUNK):
            pltpu.sync_copy(rows_v.at[0],
                            acc_sm.at[pl.ds(r0 + q * CHUNK, CHUNK)])

    pltpu.sync_copy(ei_hbm.at[0, wid], isrc_v)
    pltpu.sync_copy(ei_hbm.at[1, wid], idst_v)
    plsc.subcore_barrier()

    for b in range(NB):
        pltpu.async_copy(hs_hbm.at[isrc_v.at[b]], rows_v.at[b], gsem.at[b])

    def body(g, carry):
        for b in range(NB):
            j = g * NB + b
            pltpu.make_async_copy(hs_hbm.at[isrc_v.at[j]], rows_v.at[b],
                                  gsem.at[b]).wait()
            pltpu.async_copy(rows_v.at[b], acc_sm.at[idst_v.at[j]],
                             ssem.at[b], add=True)

        @pl.when(g < NGROUP - 1)
        def _():
            for b in range(NB):
                j = g * NB + b
                jn = j + NB
                pltpu.make_async_copy(rows_v.at[b],
                                      acc_sm.at[idst_v.at[j]],
                                      ssem.at[b]).wait()
                pltpu.async_copy(hs_hbm.at[isrc_v.at[jn]], rows_v.at[b],
                                 gsem.at[b])
        return carry

    lax.fori_loop(0, NGROUP, body, 0)
    for b in range(NB):
        j = (NGROUP - 1) * NB + b
        pltpu.make_async_copy(rows_v.at[b], acc_sm.at[idst_v.at[j]],
                              ssem.at[b]).wait()
    plsc.subcore_barrier()
    pltpu.sync_copy(acc_sm.at[pl.ds(r0, ROWS_PT)],
                    out_hbm.at[c, pl.ds(r0, ROWS_PT)])


_mp = pl.kernel(
    _mp_body,
    out_type=jax.ShapeDtypeStruct((NC, NPAD, D_HID), jnp.float32),
    mesh=_mesh(),
    scratch_types=[
        pltpu.VMEM((NCHUNK, CHUNK), jnp.int32),
        pltpu.VMEM((NCHUNK, CHUNK), jnp.int32),
        pltpu.VMEM((NB, CHUNK, D_HID), jnp.float32),
        pltpu.MemorySpace.VMEM_SHARED((NPAD, D_HID), jnp.float32),
        pltpu.SemaphoreType.DMA((NB,)),
        pltpu.SemaphoreType.DMA((NB,)),
    ],
    compiler_params=pltpu.CompilerParams(needs_layout_passes=False,
                                         use_tc_tiling_on_sc=False),
)


def _dinv_mat(dinv2, rows):
    left = lax.slice(dinv2, (0, 0), (rows, 1))
    right = lax.slice(dinv2, (0, 1), (rows, 2))
    cols = lax.broadcasted_iota(jnp.int32, (rows, 2 * D_HID), 1)
    return jnp.where(cols < D_HID, left, right)


def _tc1a_body(x_ref, w_ref, h_ref):
    h_ref[...] = jnp.dot(x_ref[...], w_ref[...],
                         preferred_element_type=jnp.float32)


_tc1a = pl.pallas_call(
    _tc1a_body,
    in_specs=[
        pl.BlockSpec((NPAIR_N, 2 * D_IN), lambda: (0, 0)),
        pl.BlockSpec((2 * D_IN, 2 * D_HID), lambda: (0, 0)),
    ],
    out_specs=pl.BlockSpec((NPAIR_N, 2 * D_HID), lambda: (0, 0)),
    out_shape=jax.ShapeDtypeStruct((NPAIR_N, 2 * D_HID), jnp.float32),
)


def _tc1b_body(h_ref, dp_ref, hs_ref, dinv_ref):
    deg = dp_ref[0] + dp_ref[1] + 1.0
    dinv2 = lax.rsqrt(deg)
    hs_ref[pl.ds(0, NPAIR_N), :] = h_ref[...] * _dinv_mat(dinv2, NPAIR_N)
    dinv_ref[...] = dinv2


_tc1b = pl.pallas_call(
    _tc1b_body,
    in_specs=[
        pl.BlockSpec((NPAIR_N, 2 * D_HID), lambda: (0, 0)),
        pl.BlockSpec((NC, NPAIR, 2), lambda: (0, 0, 0)),
    ],
    out_specs=[
        pl.BlockSpec((NPAIR, 2 * D_HID), lambda: (0, 0)),
        pl.BlockSpec((NPAIR, 2), lambda: (0, 0)),
    ],
    out_shape=[
        jax.ShapeDtypeStruct((NPAIR, 2 * D_HID), jnp.float32),
        jax.ShapeDtypeStruct((NPAIR, 2), jnp.float32),
    ],
)


def _tc2_body(acc_ref, dinv_ref, b_ref, w_ref, out_ref):
    dm = _dinv_mat(dinv_ref[...], NPAIR)
    t = jnp.maximum((acc_ref[0] + acc_ref[1]) * dm + b_ref[...], 0.0)
    out_ref[...] = jnp.dot(t, w_ref[...],
                           preferred_element_type=jnp.float32) * dm


_tc2 = pl.pallas_call(
    _tc2_body,
    in_specs=[
        pl.BlockSpec((NC, NPAIR, 2 * D_HID), lambda: (0, 0, 0)),
        pl.BlockSpec((NPAIR, 2), lambda: (0, 0)),
        pl.BlockSpec((1, 2 * D_HID), lambda: (0, 0)),
        pl.BlockSpec((2 * D_HID, 2 * D_HID), lambda: (0, 0)),
    ],
    out_specs=pl.BlockSpec((NPAIR, 2 * D_HID), lambda: (0, 0)),
    out_shape=jax.ShapeDtypeStruct((NPAIR, 2 * D_HID), jnp.float32),
)


def _tc3_body(acc_ref, dinv_ref, b_ref, w_ref, bf_ref, out_ref):
    dm = _dinv_mat(dinv_ref[...], NPAIR)
    t = jnp.maximum((acc_ref[0] + acc_ref[1]) * dm + b_ref[...], 0.0)
    out_ref[...] = jnp.dot(t, w_ref[...],
                           preferred_element_type=jnp.float32) + bf_ref[...]


_tc3 = pl.pallas_call(
    _tc3_body,
    in_specs=[
        pl.BlockSpec((NC, NPAIR, 2 * D_HID), lambda: (0, 0, 0)),
        pl.BlockSpec((NPAIR, 2), lambda: (0, 0)),
        pl.BlockSpec((1, 2 * D_HID), lambda: (0, 0)),
        pl.BlockSpec((2 * D_HID, 2 * D_OUT), lambda: (0, 0)),
        pl.BlockSpec((1, 2 * D_OUT), lambda: (0, 0)),
    ],
    out_specs=pl.BlockSpec((NPAIR, 2 * D_OUT), lambda: (0, 0)),
    out_shape=jax.ShapeDtypeStruct((NPAIR, 2 * D_OUT), jnp.float32),
)


def _blockdiag(w):
    k, m = w.shape
    z = jnp.zeros((k, m), jnp.float32)
    return jnp.concatenate(
        [jnp.concatenate([w, z], axis=1), jnp.concatenate([z, w], axis=1)],
        axis=0)


@jax.jit
def kernel(x, edge_index, W1, b1, W2, b2, Wf, bf):
    ei = edge_index.astype(jnp.int32)
    ei4 = ei.reshape(2, NW, NCHUNK, CHUNK)
    ei3 = ei.reshape(2, NW, EPW)
    zpad = jnp.zeros((NPAD,), jnp.float32)

    W1b = _blockdiag(W1)
    W2b = _blockdiag(W2)
    Wfb = _blockdiag(Wf)
    b1p = jnp.concatenate([b1, b1]).reshape(1, 2 * D_HID)
    b2p = jnp.concatenate([b2, b2]).reshape(1, 2 * D_HID)
    bfp = jnp.concatenate([bf, bf]).reshape(1, 2 * D_OUT)

    deg_parts = _deg(ei3, zpad)
    h1 = _tc1a(x.reshape(NPAIR_N, 2 * D_IN), W1b)
    hs1, dinv2 = _tc1b(h1, deg_parts.reshape(NC, NPAIR, 2))
    acc1 = _mp(hs1.reshape(NPAD, D_HID), ei4)
    hs2 = _tc2(acc1.reshape(NC, NPAIR, 2 * D_HID), dinv2, b1p, W2b)
    acc2 = _mp(hs2.reshape(NPAD, D_HID), ei4)
    outp = _tc3(acc2.reshape(NC, NPAIR, 2 * D_HID), dinv2, b2p, Wfb, bfp)
    return outp.reshape(NPAD, D_OUT)[:N]


# speedup vs baseline: 3.8042x; 1.0212x over previous
import jax
import jax.numpy as jnp
from jax import lax
from jax.experimental import pallas as pl
from jax.experimental.pallas import tpu as pltpu
from jax.experimental.pallas import tpu_sc as plsc

N = 10000
E = 320000
D_IN = 128
D_HID = 64
D_OUT = 4

NC = 2
NS = 16
NW = NC * NS
EPW = E // NW
CHUNK = 80
NCHUNK = EPW // CHUNK
NPAD = 10240
SEG = NPAD // NS
ROWS_PT = NPAD // NS
NPAIR = NPAD // 2
NPAIR_N = N // 2


def _mesh():
    return plsc.VectorSubcoreMesh(core_axis_name="c", subcore_axis_name="s")


def _deg_body(ei_hbm, zeros_hbm, deg_out, hist_v, idx_v, seg_v, stage_v,
              rep_v, shared_sm):
    c = lax.axis_index("c")
    s = lax.axis_index("s")
    wid = c * NS + s
    pltpu.sync_copy(zeros_hbm, hist_v)
    pltpu.sync_copy(ei_hbm.at[1, wid], idx_v)
    ones = jnp.full((16,), 1.0, jnp.float32)

    def body(i, carry):
        for k in range(5):
            idx = idx_v[pl.ds(i * 80 + k * 16, 16)]
            plsc.addupdate_scatter(hist_v, [idx], ones)
        return carry

    lax.fori_loop(0, EPW // 80, body, 0)

    for t in range(NS):
        pltpu.sync_copy(hist_v.at[pl.ds(t * SEG, SEG)], shared_sm.at[s, t])
    plsc.subcore_barrier()
    for t in range(NS):
        pltpu.sync_copy(shared_sm.at[t, s], seg_v.at[t])

    def red(k, carry):
        v = jnp.zeros((16,), jnp.float32)
        for t in range(NS):
            v = v + seg_v[t, pl.ds(k * 16, 16)]
        stage_v[...] = v
        for t in range(16):
            bv = plsc.load_gather(stage_v, [jnp.full((16,), t, jnp.int32)])
            for q in range(D_HID // 16):
                rep_v[k * 16 + t, pl.ds(q * 16, 16)] = bv
        return carry

    lax.fori_loop(0, SEG // 16, red, 0)
    pltpu.sync_copy(rep_v, deg_out.at[c, pl.ds(s * SEG, SEG)])


_deg = pl.kernel(
    _deg_body,
    out_type=jax.ShapeDtypeStruct((NC, NPAD, D_HID), jnp.float32),
    mesh=_mesh(),
    scratch_types=[
        pltpu.VMEM((NPAD,), jnp.float32),
        pltpu.VMEM((EPW,), jnp.int32),
        pltpu.VMEM((NS, SEG), jnp.float32),
        pltpu.VMEM((16,), jnp.float32),
        pltpu.VMEM((SEG, D_HID), jnp.float32),
        pltpu.MemorySpace.VMEM_SHARED((NS, NS, SEG), jnp.float32),
    ],
    compiler_params=pltpu.CompilerParams(needs_layout_passes=False,
                                         use_tc_tiling_on_sc=False),
)


NB = 5
NGROUP = NCHUNK // NB


def _mp_body(hs_hbm, ei_hbm, out_hbm, isrc_v, idst_v, rows_v, acc_sm,
             gsem, ssem):
    c = lax.axis_index("c")
    s = lax.axis_index("s")
    wid = c * NS + s
    r0 = s * ROWS_PT

    @pl.when(c == 0)
    def _():
        pltpu.sync_copy(hs_hbm.at[pl.ds(r0, ROWS_PT)],
                        acc_sm.at[pl.ds(r0, ROWS_PT)])

    @pl.when(c == 1)
    def _():
        z16 = jnp.zeros((16,), jnp.float32)

        def zero(i, carry):
            for k in range(D_HID // 16):
                rows_v[0, i, pl.ds(k * 16, 16)] = z16
            return carry
        lax.fori_loop(0, CHUNK, zero, 0)
        for q in range(ROWS_PT // CHUNK):
            pltpu.sync_copy(rows_v.at[0],
                            acc_sm.at[pl.ds(r0 + q * CHUNK, CHUNK)])

    pltpu.sync_copy(ei_hbm.at[0, wid], isrc_v)
    pltpu.sync_copy(ei_hbm.at[1, wid], idst_v)
    plsc.subcore_barrier()

    for b in range(NB):
        pltpu.async_copy(hs_hbm.at[isrc_v.at[b]], rows_v.at[b], gsem.at[b])

    def body(g, carry):
        for b in range(NB):
            j = g * NB + b
            pltpu.make_async_copy(hs_hbm.at[isrc_v.at[j]], rows_v.at[b],
                                  gsem.at[b]).wait()
            pltpu.async_copy(rows_v.at[b], acc_sm.at[idst_v.at[j]],
                             ssem.at[b], add=True)

        @pl.when(g < NGROUP - 1)
        def _():
            for b in range(NB):
                j = g * NB + b
                jn = j + NB
                pltpu.make_async_copy(rows_v.at[b],
                                      acc_sm.at[idst_v.at[j]],
                                      ssem.at[b]).wait()
                pltpu.async_copy(hs_hbm.at[isrc_v.at[jn]], rows_v.at[b],
                                 gsem.at[b])
        return carry

    lax.fori_loop(0, NGROUP, body, 0)
    for b in range(NB):
        j = (NGROUP - 1) * NB + b
        pltpu.make_async_copy(rows_v.at[b], acc_sm.at[idst_v.at[j]],
                              ssem.at[b]).wait()
    plsc.subcore_barrier()
    pltpu.sync_copy(acc_sm.at[pl.ds(r0, ROWS_PT)],
                    out_hbm.at[c, pl.ds(r0, ROWS_PT)])


_mp = pl.kernel(
    _mp_body,
    out_type=jax.ShapeDtypeStruct((NC, NPAD, D_HID), jnp.float32),
    mesh=_mesh(),
    scratch_types=[
        pltpu.VMEM((NCHUNK, CHUNK), jnp.int32),
        pltpu.VMEM((NCHUNK, CHUNK), jnp.int32),
        pltpu.VMEM((NB, CHUNK, D_HID), jnp.float32),
        pltpu.MemorySpace.VMEM_SHARED((NPAD, D_HID), jnp.float32),
        pltpu.SemaphoreType.DMA((NB,)),
        pltpu.SemaphoreType.DMA((NB,)),
    ],
    compiler_params=pltpu.CompilerParams(needs_layout_passes=False,
                                         use_tc_tiling_on_sc=False),
)


def _tc1a_body(x_ref, w_ref, h_ref):
    h_ref[...] = jnp.dot(x_ref[...], w_ref[...],
                         preferred_element_type=jnp.float32)


_tc1a = pl.pallas_call(
    _tc1a_body,
    in_specs=[
        pl.BlockSpec((NPAIR_N, 2 * D_IN), lambda: (0, 0)),
        pl.BlockSpec((2 * D_IN, 2 * D_HID), lambda: (0, 0)),
    ],
    out_specs=pl.BlockSpec((NPAIR_N, 2 * D_HID), lambda: (0, 0)),
    out_shape=jax.ShapeDtypeStruct((NPAIR_N, 2 * D_HID), jnp.float32),
)


def _tc1b_body(h_ref, dp_ref, hs_ref, dinv_ref):
    deg = dp_ref[0] + dp_ref[1] + 1.0
    dm = lax.rsqrt(deg)
    hs_ref[pl.ds(0, NPAIR_N), :] = h_ref[...] * lax.slice(
        dm, (0, 0), (NPAIR_N, 2 * D_HID))
    dinv_ref[...] = dm


_tc1b = pl.pallas_call(
    _tc1b_body,
    in_specs=[
        pl.BlockSpec((NPAIR_N, 2 * D_HID), lambda: (0, 0)),
        pl.BlockSpec((NC, NPAIR, 2 * D_HID), lambda: (0, 0, 0)),
    ],
    out_specs=[
        pl.BlockSpec((NPAIR, 2 * D_HID), lambda: (0, 0)),
        pl.BlockSpec((NPAIR, 2 * D_HID), lambda: (0, 0)),
    ],
    out_shape=[
        jax.ShapeDtypeStruct((NPAIR, 2 * D_HID), jnp.float32),
        jax.ShapeDtypeStruct((NPAIR, 2 * D_HID), jnp.float32),
    ],
)


def _tc2_body(acc_ref, dinv_ref, b_ref, w_ref, out_ref):
    dm = dinv_ref[...]
    t = jnp.maximum((acc_ref[0] + acc_ref[1]) * dm + b_ref[...], 0.0)
    out_ref[...] = jnp.dot(t, w_ref[...],
                           preferred_element_type=jnp.float32) * dm


_tc2 = pl.pallas_call(
    _tc2_body,
    in_specs=[
        pl.BlockSpec((NC, NPAIR, 2 * D_HID), lambda: (0, 0, 0)),
        pl.BlockSpec((NPAIR, 2 * D_HID), lambda: (0, 0)),
        pl.BlockSpec((1, 2 * D_HID), lambda: (0, 0)),
        pl.BlockSpec((2 * D_HID, 2 * D_HID), lambda: (0, 0)),
    ],
    out_specs=pl.BlockSpec((NPAIR, 2 * D_HID), lambda: (0, 0)),
    out_shape=jax.ShapeDtypeStruct((NPAIR, 2 * D_HID), jnp.float32),
)


def _tc3_body(acc_ref, dinv_ref, b_ref, w_ref, bf_ref, out_ref):
    dm = dinv_ref[...]
    t = jnp.maximum((acc_ref[0] + acc_ref[1]) * dm + b_ref[...], 0.0)
    out_ref[...] = jnp.dot(t, w_ref[...],
                           preferred_element_type=jnp.float32) + bf_ref[...]


_tc3 = pl.pallas_call(
    _tc3_body,
    in_specs=[
        pl.BlockSpec((NC, NPAIR, 2 * D_HID), lambda: (0, 0, 0)),
        pl.BlockSpec((NPAIR, 2 * D_HID), lambda: (0, 0)),
        pl.BlockSpec((1, 2 * D_HID), lambda: (0, 0)),
        pl.BlockSpec((2 * D_HID, 2 * D_OUT), lambda: (0, 0)),
        pl.BlockSpec((1, 2 * D_OUT), lambda: (0, 0)),
    ],
    out_specs=pl.BlockSpec((NPAIR, 2 * D_OUT), lambda: (0, 0)),
    out_shape=jax.ShapeDtypeStruct((NPAIR, 2 * D_OUT), jnp.float32),
)


def _blockdiag(w):
    k, m = w.shape
    z = jnp.zeros((k, m), jnp.float32)
    return jnp.concatenate(
        [jnp.concatenate([w, z], axis=1), jnp.concatenate([z, w], axis=1)],
        axis=0)


@jax.jit
def kernel(x, edge_index, W1, b1, W2, b2, Wf, bf):
    ei = edge_index.astype(jnp.int32)
    ei4 = ei.reshape(2, NW, NCHUNK, CHUNK)
    ei3 = ei.reshape(2, NW, EPW)
    zpad = jnp.zeros((NPAD,), jnp.float32)

    W1b = _blockdiag(W1)
    W2b = _blockdiag(W2)
    Wfb = _blockdiag(Wf)
    b1p = jnp.concatenate([b1, b1]).reshape(1, 2 * D_HID)
    b2p = jnp.concatenate([b2, b2]).reshape(1, 2 * D_HID)
    bfp = jnp.concatenate([bf, bf]).reshape(1, 2 * D_OUT)

    deg_parts = _deg(ei3, zpad)
    h1 = _tc1a(x.reshape(NPAIR_N, 2 * D_IN), W1b)
    hs1, dinv2 = _tc1b(h1, deg_parts.reshape(NC, NPAIR, 2 * D_HID))
    acc1 = _mp(hs1.reshape(NPAD, D_HID), ei4)
    hs2 = _tc2(acc1.reshape(NC, NPAIR, 2 * D_HID), dinv2, b1p, W2b)
    acc2 = _mp(hs2.reshape(NPAD, D_HID), ei4)
    outp = _tc3(acc2.reshape(NC, NPAIR, 2 * D_HID), dinv2, b2p, Wfb, bfp)
    return outp.reshape(NPAD, D_OUT)[:N]
